# Initial kernel scaffold; baseline (speedup 1.0000x reference)
import jax
import jax.numpy as jnp
from jax.experimental import pallas as pl


def kernel(x, cr, tables, W1, b1, W2, b2, W3, b3):
    raise NotImplementedError("write your pallas kernel here")


# speedup vs baseline: 49.0542x; 49.0542x over previous
import functools

import numpy as np
import jax
import jax.numpy as jnp
from jax import lax
from jax.experimental import pallas as pl
from jax.experimental.pallas import tpu as pltpu
from jax.experimental.pallas import tpu_sc as plsc

_L = 10
_DIM = 4
_T = 1 << 16
_BASE_RES = 16
_FINEST = 16 * 2 ** 10
_N = 524288
_HIDDEN = 64
_OUT = 13
_SCALE_MULTI = 0.5
_PER_LEVEL_SCALE = 2.0

_bg = np.exp((np.log(_FINEST) - np.log(_BASE_RES)) / (_L - 1))
_RES = [int(np.floor(_BASE_RES * _bg ** l)) for l in range(_L)]
_P1 = -1640531535
_P2 = 805459861

_NSUB = 16
_LPC = _L // 2
_HALF = _LPC * _T * _DIM
_PTS_PER_TILE = _N // _NSUB
_C = 256
_G = _C // 16
_NCHUNK = _PTS_PER_TILE // _C

_SELU_LAM = 1.0507009873554805
_SELU_ALPHA = 1.6732632423543772


def _sc_encode(xT, tabflat):
    mesh = plsc.VectorSubcoreMesh(core_axis_name="c", subcore_axis_name="s")

    @functools.partial(
        pl.kernel,
        out_type=(
            jax.ShapeDtypeStruct((_LPC * _DIM, _N), jnp.float32),
            jax.ShapeDtypeStruct((_LPC * _DIM, _N), jnp.float32),
        ),
        mesh=mesh,
        scratch_types=[
            pltpu.VMEM((3, _C), jnp.float32),
            pltpu.VMEM((_LPC * _DIM, _C), jnp.float32),
            pltpu.VMEM((_LPC * _DIM, 128), jnp.int32),
            pltpu.VMEM((_LPC * _DIM, 128), jnp.float32),
            pltpu.VMEM_SHARED((_HALF,), jnp.float32),
            pltpu.SemaphoreType.DMA,
            pltpu.SemaphoreType.DMA,
        ],
    )
    def enc(xT_hbm, tab_hbm, outA_hbm, outB_hbm, x_v, f_v, idx_v, rows_v, sp, gsem, ssem):
        cc = lax.axis_index("c")
        sid = lax.axis_index("s")
        ccz = cc == 0

        @pl.when(sid == 0)
        def _stage():
            pltpu.async_copy(
                tab_hbm.at[pl.ds(cc * _HALF, _HALF)], sp, ssem
            ).wait()

        plsc.subcore_barrier()

        tile_base = sid * _PTS_PER_TILE

        def chunk_body(ch, carry):
            base = tile_base + ch * _C
            pltpu.sync_copy(xT_hbm.at[:, pl.ds(base, _C)], x_v)

            def group_body(g, gcarry):
                off = g * 16
                xr = x_v[0, pl.ds(off, 16)]
                yr = x_v[1, pl.ds(off, 16)]
                zr = x_v[2, pl.ds(off, 16)]
                iota = lax.iota(jnp.int32, 16)

                handles = []
                sxs = []
                for ll in range(_LPC):
                    resf = jnp.where(ccz, float(_RES[ll]), float(_RES[ll + _LPC]))
                    sx = xr * resf
                    sy = yr * resf
                    sz = zr * resf
                    sxs.append((sx, sy, sz))
                    bx = sx.astype(jnp.int32)
                    by = sy.astype(jnp.int32)
                    bz = sz.astype(jnp.int32)
                    hy0 = by * _P1
                    hy1 = (by + 1) * _P1
                    hz0 = bz * _P2
                    hz1 = (bz + 1) * _P2
                    bx1 = bx + 1
                    lbase = ll * _T
                    cnum = 0
                    for i in (0, 1):
                        hx = bx1 if i else bx
                        for j in (0, 1):
                            hy = hy1 if j else hy0
                            for k in (0, 1):
                                hz = hz1 if k else hz0
                                h4 = (((hx ^ hy ^ hz) & (_T - 1)) | lbase) << 2
                                for d in range(_DIM):
                                    idx_v[ll * _DIM + d, pl.ds(cnum * 16, 16)] = h4 + d
                                cnum += 1
                    for d in range(_DIM):
                        r = ll * _DIM + d
                        handles.append(
                            pltpu.async_copy(sp.at[idx_v.at[r]], rows_v.at[r], gsem)
                        )
                for h in handles:
                    h.wait()

                for ll in range(_LPC):
                    sx, sy, sz = sxs[ll]
                    fx = sx - sx.astype(jnp.int32).astype(jnp.float32)
                    fy = sy - sy.astype(jnp.int32).astype(jnp.float32)
                    fz = sz - sz.astype(jnp.int32).astype(jnp.float32)
                    wxs = (1.0 - fx, fx)
                    wys = (1.0 - fy, fy)
                    wzs = (1.0 - fz, fz)
                    acc = [None] * _DIM
                    cnum = 0
                    for i in (0, 1):
                        for j in (0, 1):
                            wxy = wxs[i] * wys[j]
                            for k in (0, 1):
                                w = wxy * wzs[k]
                                for d in range(_DIM):
                                    rv = rows_v[ll * _DIM + d, pl.ds(cnum * 16, 16)]
                                    acc[d] = w * rv if acc[d] is None else acc[d] + w * rv
                                cnum += 1
                    for d in range(_DIM):
                        f_v[ll * _DIM + d, pl.ds(off, 16)] = acc[d]
                return gcarry

            lax.fori_loop(0, _G, group_body, 0)
            @pl.when(ccz)
            def _():
                pltpu.sync_copy(f_v, outA_hbm.at[:, pl.ds(base, _C)])

            @pl.when(jnp.logical_not(ccz))
            def _():
                pltpu.sync_copy(f_v, outB_hbm.at[:, pl.ds(base, _C)])

            return carry

        lax.fori_loop(0, _NCHUNK, chunk_body, 0)

    return enc(xT, tabflat)


def _selu(h):
    return _SELU_LAM * jnp.where(h > 0, h, _SELU_ALPHA * (jnp.exp(h) - 1.0))


def _mlp(fA, fB, cr2, W1, b1, W2, b2, W3, b3):
    B = 2048
    grid = (_N // B,)
    HR = _LPC * _DIM

    def body(fA_ref, fB_ref, cr_ref, w1_ref, b1_ref, w2_ref, b2_ref, w3_ref,
             b3_ref, o_ref):
        crv = cr_ref[...]
        nrow = lax.broadcasted_iota(jnp.int32, (HR, B), 0)
        nmod = (nrow % _L).astype(jnp.float32)
        crf = crv * _SCALE_MULTI
        inner = (_PER_LEVEL_SCALE * 4.0 * nmod) * crf
        denom = jnp.sqrt(jnp.maximum(inner, 1e-12))
        erf_x = 1.0 / jnp.maximum(denom, 1e-12)
        scaling = lax.erf(erf_x)
        w1 = w1_ref[...]
        dn = (((0,), (0,)), ((), ()))
        h = (
            lax.dot_general(fA_ref[...] * scaling, w1[:HR],
                            dn, preferred_element_type=jnp.float32)
            + lax.dot_general(fB_ref[...] * scaling, w1[HR:],
                              dn, preferred_element_type=jnp.float32)
            + b1_ref[...]
        )
        h = _selu(h)
        h = jnp.dot(h, w2_ref[...], preferred_element_type=jnp.float32) + b2_ref[...]
        h = _selu(h)
        o_ref[...] = (
            jnp.dot(h, w3_ref[...], preferred_element_type=jnp.float32) + b3_ref[...]
        )

    return pl.pallas_call(
        body,
        grid=grid,
        in_specs=[
            pl.BlockSpec((HR, B), lambda i: (0, i)),
            pl.BlockSpec((HR, B), lambda i: (0, i)),
            pl.BlockSpec((1, B), lambda i: (0, i)),
            pl.BlockSpec((_L * _DIM, _HIDDEN), lambda i: (0, 0)),
            pl.BlockSpec((1, _HIDDEN), lambda i: (0, 0)),
            pl.BlockSpec((_HIDDEN, _HIDDEN), lambda i: (0, 0)),
            pl.BlockSpec((1, _HIDDEN), lambda i: (0, 0)),
            pl.BlockSpec((_HIDDEN, _OUT), lambda i: (0, 0)),
            pl.BlockSpec((1, _OUT), lambda i: (0, 0)),
        ],
        out_specs=pl.BlockSpec((B, _OUT), lambda i: (i, 0)),
        out_shape=jax.ShapeDtypeStruct((_N, _OUT), jnp.float32),
    )(fA, fB, cr2, W1, b1, W2, b2, W3, b3)


def kernel(x, cr, tables, W1, b1, W2, b2, W3, b3):
    xT = x.T
    tabflat = tables.reshape(-1)
    fA, fB = _sc_encode(xT, tabflat)
    return _mlp(
        fA,
        fB,
        cr.reshape(1, -1),
        W1,
        b1.reshape(1, -1),
        W2,
        b2.reshape(1, -1),
        W3,
        b3.reshape(1, -1),
    )
